# Initial kernel scaffold; baseline (speedup 1.0000x reference)
import functools

import jax
import jax.numpy as jnp
from jax import lax
from jax.experimental import pallas as pl
from jax.experimental.pallas import tpu as pltpu
from jax.experimental.pallas import tpu_sc as plsc

N = 10000
E = 320000
D = 128
H = 256
C = 64

NC = 2
NS = 16
LN = 16

K = 128
TPT = ((E // NS + K - 1) // K) * K
EPAD = TPT * NS
NCHUNK = TPT // K
RPT = N // NS

_mesh = plsc.VectorSubcoreMesh(core_axis_name="c", subcore_axis_name="s")

_GDN = lax.GatherDimensionNumbers(
    offset_dims=(), collapsed_slice_dims=(0,), start_index_map=(0,))


def _splat(vec16, e):
    idx = jnp.full((LN, 1), e, dtype=jnp.int32)
    return lax.gather(vec16, idx, _GDN, slice_sizes=(1,),
                      mode=lax.GatherScatterMode.PROMISE_IN_BOUNDS)


def _zero_rows(rows_v):
    zero = jnp.zeros((LN,), jnp.float32)

    @pl.loop(0, K)
    def _(r):
        for f in range(D // LN):
            rows_v[r, pl.ds(f * LN, LN)] = zero


def _edge_pass(x_hbm, src_hbm, dst_hbm, val_hbm, src_v, dst_v, val_v,
               rows_v, acc_sh, sem, ebase):
    @pl.loop(0, NCHUNK)
    def _(ch):
        off = ebase + ch * K
        pltpu.sync_copy(src_hbm.at[pl.ds(off, K)], src_v)
        pltpu.sync_copy(dst_hbm.at[pl.ds(off, K)], dst_v)
        pltpu.sync_copy(val_hbm.at[pl.ds(off, K)], val_v)
        pltpu.async_copy(x_hbm.at[src_v], rows_v, sem).wait()

        @pl.loop(0, K // LN)
        def _(g):
            vals16 = val_v[pl.ds(g * LN, LN)]
            for e in range(LN):
                sp = _splat(vals16, e)
                r = g * LN + e
                for f in range(D // LN):
                    sl = pl.ds(f * LN, LN)
                    rows_v[r, sl] = rows_v[r, sl] * sp

        pltpu.sync_copy(rows_v, acc_sh.at[dst_v], add=True)


def _zero_acc(rows_v, acc_sh, s):
    base = s * RPT
    nfull = RPT // K
    rem = RPT - nfull * K
    for kk in range(nfull):
        pltpu.sync_copy(rows_v, acc_sh.at[pl.ds(base + kk * K, K)])
    if rem:
        pltpu.sync_copy(rows_v.at[pl.ds(0, rem)],
                        acc_sh.at[pl.ds(base + nfull * K, rem)])


_SC_SCRATCH = [
    pltpu.VMEM((K,), jnp.int32),
    pltpu.VMEM((K,), jnp.int32),
    pltpu.VMEM((K,), jnp.float32),
    pltpu.VMEM((K, D), jnp.float32),
    pltpu.VMEM_SHARED((N, D), jnp.float32),
    pltpu.SemaphoreType.DMA,
]


@functools.partial(
    pl.kernel,
    out_type=jax.ShapeDtypeStruct((NC * N, D), jnp.float32),
    mesh=_mesh,
    scratch_types=_SC_SCRATCH,
)
def _spmm_x(x_hbm, src_hbm, dst_hbm, val_hbm, out_hbm,
            src_v, dst_v, val_v, rows_v, acc_sh, sem):
    c = lax.axis_index("c")
    s = lax.axis_index("s")
    _zero_rows(rows_v)
    _zero_acc(rows_v, acc_sh, s)
    plsc.subcore_barrier()
    ebase = c * EPAD + s * TPT
    _edge_pass(x_hbm, src_hbm, dst_hbm, val_hbm, src_v, dst_v, val_v,
               rows_v, acc_sh, sem, ebase)
    plsc.subcore_barrier()
    pltpu.sync_copy(acc_sh.at[pl.ds(s * RPT, RPT)],
                    out_hbm.at[pl.ds(c * N + s * RPT, RPT)])


@functools.partial(
    pl.kernel,
    out_type=jax.ShapeDtypeStruct((2 * NC * N, D), jnp.float32),
    mesh=_mesh,
    scratch_types=_SC_SCRATCH,
)
def _spmm_h(h0_hbm, h1_hbm, src_hbm, dst_hbm, val_hbm, out_hbm,
            src_v, dst_v, val_v, rows_v, acc_sh, sem):
    c = lax.axis_index("c")
    s = lax.axis_index("s")
    ebase = c * EPAD + s * TPT
    for j, h_hbm in enumerate((h0_hbm, h1_hbm)):
        _zero_rows(rows_v)
        _zero_acc(rows_v, acc_sh, s)
        plsc.subcore_barrier()
        _edge_pass(h_hbm, src_hbm, dst_hbm, val_hbm, src_v, dst_v, val_v,
                   rows_v, acc_sh, sem, ebase)
        plsc.subcore_barrier()
        pltpu.sync_copy(acc_sh.at[pl.ds(s * RPT, RPT)],
                        out_hbm.at[pl.ds((2 * c + j) * N + s * RPT, RPT)])
        plsc.subcore_barrier()


_BM = 2500


def _mm1_body(x_ref, a_ref, b_ref, w_ref, h0_ref, h1_ref):
    acc = jnp.dot(x_ref[...], w_ref[0:D, :],
                  preferred_element_type=jnp.float32)
    acc += jnp.dot(a_ref[...], w_ref[D:2 * D, :],
                   preferred_element_type=jnp.float32)
    acc += jnp.dot(b_ref[...], w_ref[2 * D:3 * D, :],
                   preferred_element_type=jnp.float32)
    hh = jnp.maximum(acc, 0.0)
    h0_ref[...] = hh[:, 0:D]
    h1_ref[...] = hh[:, D:2 * D]


def _mm1(x, a, b, w1):
    return pl.pallas_call(
        _mm1_body,
        grid=(N // _BM,),
        in_specs=[
            pl.BlockSpec((_BM, D), lambda i: (i, 0)),
            pl.BlockSpec((_BM, D), lambda i: (i, 0)),
            pl.BlockSpec((_BM, D), lambda i: (i, 0)),
            pl.BlockSpec((3 * D, H), lambda i: (0, 0)),
        ],
        out_specs=[
            pl.BlockSpec((_BM, D), lambda i: (i, 0)),
            pl.BlockSpec((_BM, D), lambda i: (i, 0)),
        ],
        out_shape=[
            jax.ShapeDtypeStruct((N, D), jnp.float32),
            jax.ShapeDtypeStruct((N, D), jnp.float32),
        ],
    )(x, a, b, w1)


def _mm2_body(h0, h1, p0, p1, p2, p3, w_ref, o_ref):
    acc = jnp.dot(h0[...], w_ref[0:D, :], preferred_element_type=jnp.float32)
    for i, r in enumerate((h1, p0, p1, p2, p3)):
        acc += jnp.dot(r[...], w_ref[(i + 1) * D:(i + 2) * D, :],
                       preferred_element_type=jnp.float32)
    o_ref[...] = acc


def _mm2(h0, h1, p0, p1, p2, p3, w2):
    return pl.pallas_call(
        _mm2_body,
        grid=(N // _BM,),
        in_specs=[pl.BlockSpec((_BM, D), lambda i: (i, 0))] * 6
        + [pl.BlockSpec((3 * H, C), lambda i: (0, 0))],
        out_specs=pl.BlockSpec((_BM, C), lambda i: (i, 0)),
        out_shape=jax.ShapeDtypeStruct((N, C), jnp.float32),
    )(h0, h1, p0, p1, p2, p3, w2)


def _prep_edges(edge_index, values):
    pad = EPAD - E
    src = jnp.concatenate([edge_index[0], jnp.zeros((pad,), jnp.int32)])
    dst = jnp.concatenate([edge_index[1], jnp.zeros((pad,), jnp.int32)])
    val = jnp.concatenate([values, jnp.zeros((pad,), jnp.float32)])
    return src, dst, val


@jax.jit
def kernel(inputs, L_edge_index, L_values, L3_edge_index, L3_values, W1, W2):
    sL, dL, vL = _prep_edges(L_edge_index, L_values)
    sL3, dL3, vL3 = _prep_edges(L3_edge_index, L3_values)
    src_all = jnp.concatenate([sL, sL3])
    dst_all = jnp.concatenate([dL, dL3])
    val_all = jnp.concatenate([vL, vL3])

    ab = _spmm_x(inputs, src_all, dst_all, val_all)
    h0, h1 = _mm1(inputs, ab[:N], ab[N:], W1)
    cd = _spmm_h(h0, h1, src_all, dst_all, val_all)
    out = _mm2(h0, h1, cd[:N], cd[N:2 * N], cd[2 * N:3 * N], cd[3 * N:], W2)
    return out


# speedup vs baseline: 2.9148x; 2.9148x over previous
import functools

import jax
import jax.numpy as jnp
from jax import lax
from jax.experimental import pallas as pl
from jax.experimental.pallas import tpu as pltpu
from jax.experimental.pallas import tpu_sc as plsc

N = 10000
E = 320000
D = 128
H = 256
C = 64

NC = 2
NS = 16
LN = 16

K = 128
TPT = ((E // NS + K - 1) // K) * K
EPAD = TPT * NS
NCHUNK = TPT // K
RPT = 624
REM = N - NS * RPT

_mesh = plsc.VectorSubcoreMesh(core_axis_name="c", subcore_axis_name="s")

_GDN = lax.GatherDimensionNumbers(
    offset_dims=(), collapsed_slice_dims=(0,), start_index_map=(0,))


def _splat(vec16, e):
    idx = jnp.full((LN, 1), e, dtype=jnp.int32)
    return lax.gather(vec16, idx, _GDN, slice_sizes=(1,),
                      mode=lax.GatherScatterMode.PROMISE_IN_BOUNDS)


def _zero_rows(rows_v):
    zero = jnp.zeros((LN,), jnp.float32)

    @pl.loop(0, K)
    def _(r):
        for f in range(D // LN):
            rows_v[r, pl.ds(f * LN, LN)] = zero


def _edge_pass(x_hbm, src_hbm, dst_hbm, val_hbm, src_v, dst_v, val_v,
               rows_v, acc_sh, sem, ebase):
    @pl.loop(0, NCHUNK)
    def _(ch):
        off = ebase + ch * K
        pltpu.sync_copy(src_hbm.at[pl.ds(off, K)], src_v)
        pltpu.sync_copy(dst_hbm.at[pl.ds(off, K)], dst_v)
        pltpu.sync_copy(val_hbm.at[pl.ds(off, K)], val_v)
        pltpu.async_copy(x_hbm.at[src_v], rows_v, sem).wait()

        @pl.loop(0, K // LN)
        def _(g):
            vals16 = val_v[pl.ds(g * LN, LN)]
            for e in range(LN):
                sp = _splat(vals16, e)
                r = g * LN + e
                for f in range(D // LN):
                    sl = pl.ds(f * LN, LN)
                    rows_v[r, sl] = rows_v[r, sl] * sp

        pltpu.sync_copy(rows_v, acc_sh.at[dst_v], add=True)


def _zero_acc(rows_v, acc_sh, s):
    base = s * RPT
    nfull = RPT // K
    rem = RPT - nfull * K
    for kk in range(nfull):
        pltpu.sync_copy(rows_v, acc_sh.at[pl.ds(base + kk * K, K)])
    if rem:
        pltpu.sync_copy(rows_v.at[pl.ds(0, rem)],
                        acc_sh.at[pl.ds(base + nfull * K, rem)])

    @pl.when(s == NS - 1)
    def _():
        pltpu.sync_copy(rows_v.at[pl.ds(0, REM)],
                        acc_sh.at[pl.ds(NS * RPT, REM)])


def _writeout(acc_sh, out_hbm, s, out_base):
    pltpu.sync_copy(acc_sh.at[pl.ds(s * RPT, RPT)],
                    out_hbm.at[pl.ds(out_base + s * RPT, RPT)])

    @pl.when(s == NS - 1)
    def _():
        pltpu.sync_copy(acc_sh.at[pl.ds(NS * RPT, REM)],
                        out_hbm.at[pl.ds(out_base + NS * RPT, REM)])


_SC_SCRATCH = [
    pltpu.VMEM((K,), jnp.int32),
    pltpu.VMEM((K,), jnp.int32),
    pltpu.VMEM((K,), jnp.float32),
    pltpu.VMEM((K, D), jnp.float32),
    pltpu.VMEM_SHARED((N, D), jnp.float32),
    pltpu.SemaphoreType.DMA,
]


@functools.partial(
    pl.kernel,
    out_type=jax.ShapeDtypeStruct((NC * N, D), jnp.float32),
    mesh=_mesh,
    scratch_types=_SC_SCRATCH,
)
def _spmm_x(x_hbm, src_hbm, dst_hbm, val_hbm, out_hbm,
            src_v, dst_v, val_v, rows_v, acc_sh, sem):
    c = lax.axis_index("c")
    s = lax.axis_index("s")
    _zero_rows(rows_v)
    _zero_acc(rows_v, acc_sh, s)
    plsc.subcore_barrier()
    ebase = c * EPAD + s * TPT
    _edge_pass(x_hbm, src_hbm, dst_hbm, val_hbm, src_v, dst_v, val_v,
               rows_v, acc_sh, sem, ebase)
    plsc.subcore_barrier()
    _writeout(acc_sh, out_hbm, s, c * N)


@functools.partial(
    pl.kernel,
    out_type=jax.ShapeDtypeStruct((2 * NC * N, D), jnp.float32),
    mesh=_mesh,
    scratch_types=_SC_SCRATCH,
)
def _spmm_h(h0_hbm, h1_hbm, src_hbm, dst_hbm, val_hbm, out_hbm,
            src_v, dst_v, val_v, rows_v, acc_sh, sem):
    c = lax.axis_index("c")
    s = lax.axis_index("s")
    ebase = c * EPAD + s * TPT
    for j, h_hbm in enumerate((h0_hbm, h1_hbm)):
        _zero_rows(rows_v)
        _zero_acc(rows_v, acc_sh, s)
        plsc.subcore_barrier()
        _edge_pass(h_hbm, src_hbm, dst_hbm, val_hbm, src_v, dst_v, val_v,
                   rows_v, acc_sh, sem, ebase)
        plsc.subcore_barrier()
        _writeout(acc_sh, out_hbm, s, (2 * c + j) * N)
        plsc.subcore_barrier()


_BM = 2000


def _mm1_body(x_ref, a_ref, b_ref, w_ref, h0_ref, h1_ref):
    acc = jnp.dot(x_ref[...], w_ref[0:D, :],
                  preferred_element_type=jnp.float32)
    acc += jnp.dot(a_ref[...], w_ref[D:2 * D, :],
                   preferred_element_type=jnp.float32)
    acc += jnp.dot(b_ref[...], w_ref[2 * D:3 * D, :],
                   preferred_element_type=jnp.float32)
    hh = jnp.maximum(acc, 0.0)
    h0_ref[...] = hh[:, 0:D]
    h1_ref[...] = hh[:, D:2 * D]


def _mm1(x, a, b, w1):
    return pl.pallas_call(
        _mm1_body,
        grid=(N // _BM,),
        in_specs=[
            pl.BlockSpec((_BM, D), lambda i: (i, 0)),
            pl.BlockSpec((_BM, D), lambda i: (i, 0)),
            pl.BlockSpec((_BM, D), lambda i: (i, 0)),
            pl.BlockSpec((3 * D, H), lambda i: (0, 0)),
        ],
        out_specs=[
            pl.BlockSpec((_BM, D), lambda i: (i, 0)),
            pl.BlockSpec((_BM, D), lambda i: (i, 0)),
        ],
        out_shape=[
            jax.ShapeDtypeStruct((N, D), jnp.float32),
            jax.ShapeDtypeStruct((N, D), jnp.float32),
        ],
    )(x, a, b, w1)


def _mm2_body(h0, h1, p0, p1, p2, p3, w_ref, o_ref):
    acc = jnp.dot(h0[...], w_ref[0:D, :], preferred_element_type=jnp.float32)
    for i, r in enumerate((h1, p0, p1, p2, p3)):
        acc += jnp.dot(r[...], w_ref[(i + 1) * D:(i + 2) * D, :],
                       preferred_element_type=jnp.float32)
    o_ref[...] = acc


def _mm2(h0, h1, p0, p1, p2, p3, w2):
    return pl.pallas_call(
        _mm2_body,
        grid=(N // _BM,),
        in_specs=[pl.BlockSpec((_BM, D), lambda i: (i, 0))] * 6
        + [pl.BlockSpec((3 * H, C), lambda i: (0, 0))],
        out_specs=pl.BlockSpec((_BM, C), lambda i: (i, 0)),
        out_shape=jax.ShapeDtypeStruct((N, C), jnp.float32),
    )(h0, h1, p0, p1, p2, p3, w2)


def _prep_edges(edge_index, values):
    pad = EPAD - E
    src = jnp.concatenate([edge_index[0], jnp.zeros((pad,), jnp.int32)])
    dst = jnp.concatenate([edge_index[1], jnp.zeros((pad,), jnp.int32)])
    val = jnp.concatenate([values, jnp.zeros((pad,), jnp.float32)])
    return src, dst, val


@jax.jit
def kernel(inputs, L_edge_index, L_values, L3_edge_index, L3_values, W1, W2):
    sL, dL, vL = _prep_edges(L_edge_index, L_values)
    sL3, dL3, vL3 = _prep_edges(L3_edge_index, L3_values)
    src_all = jnp.concatenate([sL, sL3])
    dst_all = jnp.concatenate([dL, dL3])
    val_all = jnp.concatenate([vL, vL3])

    ab = _spmm_x(inputs, src_all, dst_all, val_all)
    h0, h1 = _mm1(inputs, ab[:N], ab[N:], W1)
    cd = _spmm_h(h0, h1, src_all, dst_all, val_all)
    out = _mm2(h0, h1, cd[:N], cd[N:2 * N], cd[2 * N:3 * N], cd[3 * N:], W2)
    return out
